# baseline (speedup 1.0000x reference)
import jax
import jax.numpy as jnp
from jax.experimental import pallas as pl
from jax.experimental.pallas import tpu as pltpu

_INPUT_NUM = 4
_TIME_STEP = 5
_INPUT_DIM = 10
_HID = 128
_N_STOCK = 100
_N_CAT = 5
_GROUP = 20


def _leaky(x):
    return jnp.where(x >= 0, x, 0.2 * x)


def _sigmoid(x):
    return 0.5 * jnp.tanh(0.5 * x) + 0.5


def _body(
    xw_ref, eWih_ref, eWhh_ref, eBih_ref, eBhh_ref,
    eAttW_ref, eAttB_ref,
    wkWih_ref, wkWhh_ref, wkBih_ref, wkBhh_ref,
    wattW_ref, wattB_ref,
    poolW_ref, poolB_ref,
    innerW_ref, iSrc_ref, iDst_ref, iBias_ref,
    catW_ref, cSrc_ref, cDst_ref, cBias_ref,
    fW_ref, fB_ref,
    regW_ref, regB_ref, clsW_ref, clsB_ref,
    reg_ref, cls_ref,
    gi2_scr, ys2_scr,
):
    f32 = jnp.float32
    H = _HID
    G3 = 3 * H
    wk_ih_t = wkWih_ref[:].T
    wk_b_ih = wkBih_ref[:]
    wk_b_hh = wkBhh_ref[:]
    lane = jax.lax.broadcasted_iota(jnp.int32, (1, G3), 1)
    wk_b2 = wk_b_ih + jnp.where(lane < 2 * H, wk_b_hh, 0.0)

    for wi in range(_INPUT_NUM):
        x = xw_ref[wi]
        w_ih_t = eWih_ref[wi].T
        w_hh_t = eWhh_ref[wi].T
        b_ih = eBih_ref[wi]
        b_hh = eBhh_ref[wi]

        h = jnp.zeros((_N_STOCK, H), f32)
        ys = []
        for t in range(_TIME_STEP):
            x_t = x[:, _INPUT_DIM * t:_INPUT_DIM * (t + 1)]
            gi = jnp.dot(x_t, w_ih_t, preferred_element_type=f32) + b_ih
            gh = jnp.dot(h, w_hh_t, preferred_element_type=f32) + b_hh
            r = _sigmoid(gi[:, 0:H] + gh[:, 0:H])
            z = _sigmoid(gi[:, H:2 * H] + gh[:, H:2 * H])
            n = jnp.tanh(gi[:, 2 * H:G3] + r * gh[:, 2 * H:G3])
            h = n + z * (h - n)
            ys.append(h)

        aw = []
        for tp in range(_TIME_STEP):
            acc = ys[0] * eAttW_ref[wi, tp, 0]
            for t in range(1, _TIME_STEP):
                acc = acc + ys[t] * eAttW_ref[wi, tp, t]
            aw.append(acc + eAttB_ref[wi, tp])
        m = aw[0]
        for tp in range(1, _TIME_STEP):
            m = jnp.maximum(m, aw[tp])
        es = [jnp.exp(a - m) for a in aw]
        den = es[0]
        for tp in range(1, _TIME_STEP):
            den = den + es[tp]
        inv = 1.0 / den
        av = es[0] * ys[0]
        for tp in range(1, _TIME_STEP):
            av = av + es[tp] * ys[tp]
        av = av * inv

        gi2 = jnp.dot(av, wk_ih_t, preferred_element_type=f32) + wk_b2
        gi2_scr[:, wi, :] = gi2

    w_hh2 = wkWhh_ref[:].T
    b_hhn = wk_b_hh[2 * H:G3]

    def step(t, h):
        gi = gi2_scr[pl.ds(t, 1)].reshape(_INPUT_NUM, G3)
        gh = jnp.dot(h, w_hh2, preferred_element_type=f32)
        r = _sigmoid(gi[:, 0:H] + gh[:, 0:H])
        z = _sigmoid(gi[:, H:2 * H] + gh[:, H:2 * H])
        n = jnp.tanh(gi[:, 2 * H:G3] + r * (gh[:, 2 * H:G3] + b_hhn))
        h_new = n + z * (h - n)
        ys2_scr[pl.ds(t, 1)] = h_new[None]
        return h_new

    jax.lax.fori_loop(0, _N_STOCK, step, jnp.zeros((_INPUT_NUM, H), f32),
                      unroll=4)

    ys2 = ys2_scr[:]
    yt = [ys2[:, t, :] for t in range(_INPUT_NUM)]
    aw2 = []
    for tp in range(_INPUT_NUM):
        acc = yt[0] * wattW_ref[tp, 0]
        for t in range(1, _INPUT_NUM):
            acc = acc + yt[t] * wattW_ref[tp, t]
        aw2.append(acc + wattB_ref[tp])
    m2 = jnp.maximum(jnp.maximum(aw2[0], aw2[1]), jnp.maximum(aw2[2], aw2[3]))
    e2 = [jnp.exp(a - m2) for a in aw2]
    den2 = (e2[0] + e2[1]) + (e2[2] + e2[3])
    wav = (e2[0] * yt[0] + e2[1] * yt[1] + e2[2] * yt[2] + e2[3] * yt[3]) / den2

    poolW = poolW_ref[:]
    poolB = poolB_ref[:].reshape(_GROUP, 1)
    cv_rows = []
    for c in range(_N_CAT):
        w_c = wav[c * _GROUP:(c + 1) * _GROUP]
        a_c = jnp.dot(poolW, w_c, preferred_element_type=f32) + poolB
        m_c = jnp.max(a_c, axis=0, keepdims=True)
        e_c = jnp.exp(a_c - m_c)
        den_c = jnp.sum(e_c, axis=0, keepdims=True)
        cv_rows.append(jnp.sum(e_c * w_c, axis=0, keepdims=True) / den_c)
    cv = jnp.concatenate(cv_rows, axis=0)

    h1 = jax.lax.dot_general(wav, innerW_ref[:], (((1,), (1,)), ((), ())),
                             preferred_element_type=f32)
    a_src = jnp.sum(h1 * iSrc_ref[:], axis=1, keepdims=True)
    a_dst = jnp.sum(h1 * iDst_ref[:], axis=1, keepdims=True)
    parts_h, parts_a = [], []
    for c in range(_N_CAT):
        lo = c * _GROUP
        parts_h.append(h1[lo + _GROUP - 1:lo + _GROUP])
        parts_h.append(h1[lo:lo + _GROUP - 1])
        parts_a.append(a_src[lo + _GROUP - 1:lo + _GROUP])
        parts_a.append(a_src[lo:lo + _GROUP - 1])
    h1_prev = jnp.concatenate(parts_h, axis=0)
    a_src_prev = jnp.concatenate(parts_a, axis=0)
    al_self = _leaky(a_src + a_dst)
    al_prev = _leaky(a_src_prev + a_dst)
    m_i = jnp.maximum(al_self, al_prev)
    e_self = jnp.exp(al_self - m_i)
    e_prev = jnp.exp(al_prev - m_i)
    inv_i = 1.0 / (e_self + e_prev + 1e-16)
    inner_emb = (e_self * h1 + e_prev * h1_prev) * inv_i + iBias_ref[:]

    h2 = jax.lax.dot_general(cv, catW_ref[:], (((1,), (1,)), ((), ())),
                             preferred_element_type=f32)
    as_row = jax.lax.dot_general(cSrc_ref[:].reshape(1, H), h2,
                                 (((1,), (1,)), ((), ())),
                                 preferred_element_type=f32)
    ad_col = jnp.sum(h2 * cDst_ref[:], axis=1, keepdims=True)
    alpha = _leaky(ad_col + as_row)
    m_o = jnp.max(alpha, axis=1, keepdims=True)
    e_o = jnp.exp(alpha - m_o)
    den_o = jnp.sum(e_o, axis=1, keepdims=True) + 1e-16
    coef = e_o / den_o
    cat_out = jnp.dot(coef, h2, preferred_element_type=f32) + cBias_ref[:]

    fW = fW_ref[:]
    cat_fused = jax.lax.dot_general(
        cat_out, fW[:, H:2 * H], (((1,), (1,)), ((), ())),
        preferred_element_type=f32)
    cat_bcast = jnp.concatenate(
        [jnp.broadcast_to(cat_fused[c:c + 1], (_GROUP, H)) for c in range(_N_CAT)],
        axis=0)
    fv = (jax.lax.dot_general(wav, fW[:, 0:H], (((1,), (1,)), ((), ())),
                              preferred_element_type=f32)
          + cat_bcast
          + jax.lax.dot_general(inner_emb, fW[:, 2 * H:G3],
                                (((1,), (1,)), ((), ())),
                                preferred_element_type=f32)
          + fB_ref[:])
    fv = jnp.maximum(fv, 0.0)
    reg_row = jax.lax.dot_general(regW_ref[:], fv, (((1,), (1,)), ((), ())),
                                  preferred_element_type=f32) + regB_ref[0]
    cls_row = _sigmoid(
        jax.lax.dot_general(clsW_ref[:], fv, (((1,), (1,)), ((), ())),
                            preferred_element_type=f32) + clsB_ref[0])
    reg_ref[:] = reg_row.reshape(_N_STOCK)
    cls_ref[:] = cls_row.reshape(_N_STOCK)


def kernel(weekly_batch, enc_W_ih, enc_W_hh, enc_b_ih, enc_b_hh, enc_att_w,
           enc_att_b, wk_W_ih, wk_W_hh, wk_b_ih, wk_b_hh, watt_w, watt_b,
           pool_w, pool_b, inner_W, inner_att_src, inner_att_dst, inner_bias,
           cat_W, cat_att_src, cat_att_dst, cat_bias, fusion_w, fusion_b,
           reg_w, reg_b, cls_w, cls_b, inner_edge, outer_edge):
    f32 = jnp.float32
    H = _HID
    xw = weekly_batch.reshape(_INPUT_NUM, _N_STOCK,
                              _TIME_STEP * _INPUT_DIM)
    vmem = pl.BlockSpec(memory_space=pltpu.VMEM)
    smem = pl.BlockSpec(memory_space=pltpu.SMEM)
    in_specs = ([vmem] * 5 + [smem, smem] + [vmem] * 4 + [smem, smem]
                + [vmem] * 12 + [vmem, smem, vmem, smem])

    reg, cls = pl.pallas_call(
        _body,
        out_shape=[jax.ShapeDtypeStruct((_N_STOCK,), f32),
                   jax.ShapeDtypeStruct((_N_STOCK,), f32)],
        in_specs=in_specs,
        out_specs=[vmem, vmem],
        scratch_shapes=[
            pltpu.VMEM((_N_STOCK, _INPUT_NUM, 3 * H), f32),
            pltpu.VMEM((_N_STOCK, _INPUT_NUM, H), f32),
        ],
    )(
        xw,
        enc_W_ih, enc_W_hh,
        enc_b_ih, enc_b_hh,
        enc_att_w, enc_att_b,
        wk_W_ih, wk_W_hh,
        wk_b_ih, wk_b_hh,
        watt_w, watt_b,
        pool_w, pool_b,
        inner_W, inner_att_src, inner_att_dst, inner_bias,
        cat_W, cat_att_src, cat_att_dst, cat_bias,
        fusion_w, fusion_b,
        reg_w, reg_b,
        cls_w, cls_b,
    )
    return (reg, cls)


# speedup vs baseline: 1.0068x; 1.0068x over previous
import jax
import jax.numpy as jnp
from jax.experimental import pallas as pl
from jax.experimental.pallas import tpu as pltpu

_INPUT_NUM = 4
_TIME_STEP = 5
_INPUT_DIM = 10
_HID = 128
_N_STOCK = 100
_N_CAT = 5
_GROUP = 20


def _leaky(x):
    return jnp.where(x >= 0, x, 0.2 * x)


def _sigmoid(x):
    return 0.5 * jnp.tanh(0.5 * x) + 0.5


def _body(
    xw_ref, eWih_ref, eWhh_ref, eBih_ref, eBhh_ref,
    eAttW_ref, eAttB_ref,
    wkWih_ref, wkWhh_ref, wkBih_ref, wkBhh_ref,
    wattW_ref, wattB_ref,
    poolW_ref, poolB_ref,
    innerW_ref, iSrc_ref, iDst_ref, iBias_ref,
    catW_ref, cSrc_ref, cDst_ref, cBias_ref,
    fW_ref, fB_ref,
    regW_ref, regB_ref, clsW_ref, clsB_ref,
    reg_ref, cls_ref,
    gi2_scr, ys2_scr,
):
    f32 = jnp.float32
    H = _HID
    G3 = 3 * H
    wk_ih_t = wkWih_ref[:].T
    wk_b_ih = wkBih_ref[:]
    wk_b_hh = wkBhh_ref[:]
    lane = jax.lax.broadcasted_iota(jnp.int32, (1, G3), 1)
    wk_b2 = wk_b_ih + jnp.where(lane < 2 * H, wk_b_hh, 0.0)

    for wi in range(_INPUT_NUM):
        x = xw_ref[wi]
        w_ih_t = eWih_ref[wi].T
        w_hh_t = eWhh_ref[wi].T
        b_ih = eBih_ref[wi]
        b_hh = eBhh_ref[wi]

        h = jnp.zeros((_N_STOCK, H), f32)
        ys = []
        for t in range(_TIME_STEP):
            x_t = x[:, _INPUT_DIM * t:_INPUT_DIM * (t + 1)]
            gi = jnp.dot(x_t, w_ih_t, preferred_element_type=f32) + b_ih
            gh = jnp.dot(h, w_hh_t, preferred_element_type=f32) + b_hh
            r = _sigmoid(gi[:, 0:H] + gh[:, 0:H])
            z = _sigmoid(gi[:, H:2 * H] + gh[:, H:2 * H])
            n = jnp.tanh(gi[:, 2 * H:G3] + r * gh[:, 2 * H:G3])
            h = n + z * (h - n)
            ys.append(h)

        aw = []
        for tp in range(_TIME_STEP):
            acc = ys[0] * eAttW_ref[wi, tp, 0]
            for t in range(1, _TIME_STEP):
                acc = acc + ys[t] * eAttW_ref[wi, tp, t]
            aw.append(acc + eAttB_ref[wi, tp])
        m = aw[0]
        for tp in range(1, _TIME_STEP):
            m = jnp.maximum(m, aw[tp])
        es = [jnp.exp(a - m) for a in aw]
        den = es[0]
        for tp in range(1, _TIME_STEP):
            den = den + es[tp]
        inv = 1.0 / den
        av = es[0] * ys[0]
        for tp in range(1, _TIME_STEP):
            av = av + es[tp] * ys[tp]
        av = av * inv

        gi2 = jnp.dot(av, wk_ih_t, preferred_element_type=f32) + wk_b2
        gi2_scr[:, wi, :] = gi2

    w_hh2 = wkWhh_ref[:].T
    b_hhn = wk_b_hh[2 * H:G3]

    def step(t, h):
        gi = gi2_scr[pl.ds(t, 1)].reshape(_INPUT_NUM, G3)
        gh = jnp.dot(h, w_hh2, preferred_element_type=f32)
        r = _sigmoid(gi[:, 0:H] + gh[:, 0:H])
        z = _sigmoid(gi[:, H:2 * H] + gh[:, H:2 * H])
        n = jnp.tanh(gi[:, 2 * H:G3] + r * (gh[:, 2 * H:G3] + b_hhn))
        h_new = n + z * (h - n)
        ys2_scr[pl.ds(t, 1)] = h_new[None]
        return h_new

    jax.lax.fori_loop(0, _N_STOCK, step, jnp.zeros((_INPUT_NUM, H), f32),
                      unroll=4)

    ys2 = ys2_scr[:]
    yt = [ys2[:, t, :] for t in range(_INPUT_NUM)]
    aw2 = []
    for tp in range(_INPUT_NUM):
        acc = yt[0] * wattW_ref[tp, 0]
        for t in range(1, _INPUT_NUM):
            acc = acc + yt[t] * wattW_ref[tp, t]
        aw2.append(acc + wattB_ref[tp])
    m2 = jnp.maximum(jnp.maximum(aw2[0], aw2[1]), jnp.maximum(aw2[2], aw2[3]))
    e2 = [jnp.exp(a - m2) for a in aw2]
    den2 = (e2[0] + e2[1]) + (e2[2] + e2[3])
    wav = (e2[0] * yt[0] + e2[1] * yt[1] + e2[2] * yt[2] + e2[3] * yt[3]) / den2

    poolW = poolW_ref[:]
    poolB = poolB_ref[:].reshape(_GROUP, 1)
    cv_rows = []
    for c in range(_N_CAT):
        w_c = wav[c * _GROUP:(c + 1) * _GROUP]
        a_c = jnp.dot(poolW, w_c, preferred_element_type=f32) + poolB
        m_c = jnp.max(a_c, axis=0, keepdims=True)
        e_c = jnp.exp(a_c - m_c)
        den_c = jnp.sum(e_c, axis=0, keepdims=True)
        cv_rows.append(jnp.sum(e_c * w_c, axis=0, keepdims=True) / den_c)
    cv = jnp.concatenate(cv_rows, axis=0)

    h1 = jax.lax.dot_general(wav, innerW_ref[:], (((1,), (1,)), ((), ())),
                             preferred_element_type=f32)
    a_src = jnp.sum(h1 * iSrc_ref[:], axis=1, keepdims=True)
    a_dst = jnp.sum(h1 * iDst_ref[:], axis=1, keepdims=True)
    parts_h, parts_a = [], []
    for c in range(_N_CAT):
        lo = c * _GROUP
        parts_h.append(h1[lo + _GROUP - 1:lo + _GROUP])
        parts_h.append(h1[lo:lo + _GROUP - 1])
        parts_a.append(a_src[lo + _GROUP - 1:lo + _GROUP])
        parts_a.append(a_src[lo:lo + _GROUP - 1])
    h1_prev = jnp.concatenate(parts_h, axis=0)
    a_src_prev = jnp.concatenate(parts_a, axis=0)
    al_self = _leaky(a_src + a_dst)
    al_prev = _leaky(a_src_prev + a_dst)
    m_i = jnp.maximum(al_self, al_prev)
    e_self = jnp.exp(al_self - m_i)
    e_prev = jnp.exp(al_prev - m_i)
    inv_i = 1.0 / (e_self + e_prev + 1e-16)
    inner_emb = (e_self * h1 + e_prev * h1_prev) * inv_i + iBias_ref[:]

    h2 = jax.lax.dot_general(cv, catW_ref[:], (((1,), (1,)), ((), ())),
                             preferred_element_type=f32)
    fW = fW_ref[:]
    M2 = jax.lax.dot_general(h2, fW[:, H:2 * H], (((1,), (1,)), ((), ())),
                             preferred_element_type=f32)
    cb_f = jax.lax.dot_general(cBias_ref[:].reshape(1, H), fW[:, H:2 * H],
                               (((1,), (1,)), ((), ())),
                               preferred_element_type=f32)
    as_col = jnp.sum(h2 * cSrc_ref[:], axis=1, keepdims=True)
    as_row = as_col.T
    ad_col = jnp.sum(h2 * cDst_ref[:], axis=1, keepdims=True)
    alpha = _leaky(ad_col + as_row)
    m_o = jnp.max(alpha, axis=1, keepdims=True)
    e_o = jnp.exp(alpha - m_o)
    den_o = jnp.sum(e_o, axis=1, keepdims=True) + 1e-16
    coef = e_o / den_o

    cat_fused = jnp.dot(coef, M2, preferred_element_type=f32) + cb_f
    cat_bcast = jnp.concatenate(
        [jnp.broadcast_to(cat_fused[c:c + 1], (_GROUP, H)) for c in range(_N_CAT)],
        axis=0)
    fv = (jax.lax.dot_general(wav, fW[:, 0:H], (((1,), (1,)), ((), ())),
                              preferred_element_type=f32)
          + cat_bcast
          + jax.lax.dot_general(inner_emb, fW[:, 2 * H:G3],
                                (((1,), (1,)), ((), ())),
                                preferred_element_type=f32)
          + fB_ref[:])
    fv = jnp.maximum(fv, 0.0)
    headW = jnp.concatenate([regW_ref[:], clsW_ref[:]], axis=0)
    rc = jax.lax.dot_general(headW, fv, (((1,), (1,)), ((), ())),
                             preferred_element_type=f32)
    reg_ref[:] = (rc[0:1] + regB_ref[0]).reshape(_N_STOCK)
    cls_ref[:] = _sigmoid(rc[1:2] + clsB_ref[0]).reshape(_N_STOCK)


def kernel(weekly_batch, enc_W_ih, enc_W_hh, enc_b_ih, enc_b_hh, enc_att_w,
           enc_att_b, wk_W_ih, wk_W_hh, wk_b_ih, wk_b_hh, watt_w, watt_b,
           pool_w, pool_b, inner_W, inner_att_src, inner_att_dst, inner_bias,
           cat_W, cat_att_src, cat_att_dst, cat_bias, fusion_w, fusion_b,
           reg_w, reg_b, cls_w, cls_b, inner_edge, outer_edge):
    f32 = jnp.float32
    H = _HID
    xw = weekly_batch.reshape(_INPUT_NUM, _N_STOCK,
                              _TIME_STEP * _INPUT_DIM)
    vmem = pl.BlockSpec(memory_space=pltpu.VMEM)
    smem = pl.BlockSpec(memory_space=pltpu.SMEM)
    in_specs = ([vmem] * 5 + [smem, smem] + [vmem] * 4 + [smem, smem]
                + [vmem] * 12 + [vmem, smem, vmem, smem])

    reg, cls = pl.pallas_call(
        _body,
        out_shape=[jax.ShapeDtypeStruct((_N_STOCK,), f32),
                   jax.ShapeDtypeStruct((_N_STOCK,), f32)],
        in_specs=in_specs,
        out_specs=[vmem, vmem],
        scratch_shapes=[
            pltpu.VMEM((_N_STOCK, _INPUT_NUM, 3 * H), f32),
            pltpu.VMEM((_N_STOCK, _INPUT_NUM, H), f32),
        ],
    )(
        xw,
        enc_W_ih, enc_W_hh,
        enc_b_ih, enc_b_hh,
        enc_att_w, enc_att_b,
        wk_W_ih, wk_W_hh,
        wk_b_ih, wk_b_hh,
        watt_w, watt_b,
        pool_w, pool_b,
        inner_W, inner_att_src, inner_att_dst, inner_bias,
        cat_W, cat_att_src, cat_att_dst, cat_bias,
        fusion_w, fusion_b,
        reg_w, reg_b,
        cls_w, cls_b,
    )
    return (reg, cls)
